# baseline (speedup 1.0000x reference)
import functools

import jax
import jax.numpy as jnp
from jax import lax
from jax.experimental import pallas as pl
from jax.experimental.pallas import tpu as pltpu
from jax.experimental.pallas import tpu_sc as plsc

_HEADS = 16
_N = 32
_R = 2 * _N - 1
_L = 16
_NC = 2


def _bias_body(rht_hbm, rwt_hbm, out_hbm, rht_v, rwt_v, out_v):
    wid = lax.axis_index("s") * _NC + lax.axis_index("c")

    pltpu.sync_copy(rht_hbm, rht_v)
    pltpu.sync_copy(rwt_hbm, rwt_v)

    for h in range(_HEADS):
        for c in range(2):
            rv = rwt_v[pl.ds(h * 64 + 16 * c + (_N - 1) - wid, _L)]
            for i in range(_N):
                bh = rht_v[pl.ds(h * 64 + 16 * c + (_N - 1) - i, _L)]
                out_v[pl.ds((h * _N + i) * _N + 16 * c, _L)] = bh + rv

    pltpu.sync_copy(out_v, out_hbm.at[wid])


_bias_kernel = functools.partial(
    pl.kernel,
    mesh=plsc.VectorSubcoreMesh(core_axis_name="c", subcore_axis_name="s"),
    out_type=jax.ShapeDtypeStruct((_N, _HEADS * _N * _N), jnp.float32),
    scratch_types=[
        pltpu.VMEM((_HEADS * 64,), jnp.float32),
        pltpu.VMEM((_HEADS * 64,), jnp.float32),
        pltpu.VMEM((_HEADS * _N * _N,), jnp.float32),
    ],
)(_bias_body)


def kernel(rel_height, rel_width, H, W):
    del H, W
    pad = jnp.zeros((_HEADS, 1), jnp.float32)
    rht = jnp.concatenate([rel_height.T, pad], axis=1).reshape(-1)
    rwt = jnp.concatenate([rel_width[::-1].T, pad], axis=1).reshape(-1)
    out = _bias_kernel(rht, rwt)
    return out.reshape(_N, _HEADS, _N, _N)


# speedup vs baseline: 1.0289x; 1.0289x over previous
import functools

import jax
import jax.numpy as jnp
from jax import lax
from jax.experimental import pallas as pl
from jax.experimental.pallas import tpu as pltpu
from jax.experimental.pallas import tpu_sc as plsc

_HEADS = 16
_N = 32
_R = 2 * _N - 1
_L = 16
_NC = 2
_HALF = _HEADS * _N * _N // 2


def _bias_body(rht_hbm, rwt_hbm, out_hbm, rht_v, rwt_v, out_v, sem1, sem2):
    wid = lax.axis_index("s") * _NC + lax.axis_index("c")

    in1 = pltpu.async_copy(rht_hbm, rht_v, sem1)
    in2 = pltpu.async_copy(rwt_hbm, rwt_v, sem2)
    in1.wait()
    in2.wait()

    def half(h0):
        for h in range(h0, h0 + _HEADS // 2):
            for c in range(2):
                rv = rwt_v[pl.ds(h * 64 + 16 * c + (_N - 1) - wid, _L)]
                for i in range(_N):
                    bh = rht_v[pl.ds(h * 64 + 16 * c + (_N - 1) - i, _L)]
                    out_v[pl.ds((h * _N + i) * _N + 16 * c, _L)] = bh + rv

    half(0)
    out1 = pltpu.async_copy(
        out_v.at[pl.ds(0, _HALF)], out_hbm.at[wid, pl.ds(0, _HALF)], sem1)
    half(_HEADS // 2)
    out2 = pltpu.async_copy(
        out_v.at[pl.ds(_HALF, _HALF)], out_hbm.at[wid, pl.ds(_HALF, _HALF)], sem2)
    out1.wait()
    out2.wait()


_bias_kernel = functools.partial(
    pl.kernel,
    mesh=plsc.VectorSubcoreMesh(core_axis_name="c", subcore_axis_name="s"),
    out_type=jax.ShapeDtypeStruct((_N, _HEADS * _N * _N), jnp.float32),
    scratch_types=[
        pltpu.VMEM((_HEADS * 64,), jnp.float32),
        pltpu.VMEM((_HEADS * 64,), jnp.float32),
        pltpu.VMEM((_HEADS * _N * _N,), jnp.float32),
        pltpu.SemaphoreType.DMA,
        pltpu.SemaphoreType.DMA,
    ],
)(_bias_body)


def kernel(rel_height, rel_width, H, W):
    del H, W
    pad = jnp.zeros((_HEADS, 1), jnp.float32)
    rht = jnp.concatenate([rel_height.T, pad], axis=1).reshape(-1)
    rwt = jnp.concatenate([rel_width[::-1].T, pad], axis=1).reshape(-1)
    out = _bias_kernel(rht, rwt)
    return out.reshape(_N, _HEADS, _N, _N)
